# baseline (speedup 1.0000x reference)
import functools

import jax
import jax.numpy as jnp
from jax.experimental import pallas as pl
from jax.experimental.pallas import tpu as pltpu

_BLK = 512


def _prep_body(a_ref, ahat_ref, dinv_ref, deg_acc):
    i = pl.program_id(0)
    blk, n = a_ref.shape
    a = a_ref[...]
    rows = jax.lax.broadcasted_iota(jnp.int32, (blk, n), 0) + i * blk
    cols = jax.lax.broadcasted_iota(jnp.int32, (blk, n), 1)
    ahat = jnp.where(rows == cols, 1, a).astype(jnp.bfloat16)
    ahat_ref[...] = ahat
    ones = jnp.ones((blk, 1), jnp.bfloat16)
    part = jax.lax.dot_general(
        ahat, ones, (((0,), (0,)), ((), ())),
        preferred_element_type=jnp.float32)

    @pl.when(i == 0)
    def _():
        deg_acc[...] = part

    @pl.when(i > 0)
    def _():
        deg_acc[...] += part

    @pl.when(i == pl.num_programs(0) - 1)
    def _():
        deg = deg_acc[...]
        dinv_ref[...] = jnp.where(deg > 0, jax.lax.rsqrt(deg), 0.0)


def _prop_body(h_ref, w_ref, b_ref, dinv_ref, ahat_ref, out_ref, g_ref, acc_ref):
    i = pl.program_id(0)
    blk = ahat_ref.shape[0]

    @pl.when(i == 0)
    def _():
        hw = jnp.dot(h_ref[...], w_ref[...], preferred_element_type=jnp.float32)
        g_ref[...] = (dinv_ref[...] * hw).astype(jnp.bfloat16)

    g_blk = g_ref[pl.ds(i * blk, blk), :]
    part = jax.lax.dot_general(
        ahat_ref[...], g_blk, (((0,), (0,)), ((), ())),
        preferred_element_type=jnp.float32)

    @pl.when(i == 0)
    def _():
        acc_ref[...] = part

    @pl.when(i > 0)
    def _():
        acc_ref[...] += part

    @pl.when(i == pl.num_programs(0) - 1)
    def _():
        out_ref[...] = jnp.maximum(acc_ref[...] * dinv_ref[...] + b_ref[...], 0.0)


@functools.partial(jax.jit, static_argnames=("interpret",))
def _gcn_block(x, adj_matrix, W1, b1, W2, b2, interpret=False):
    n, d = x.shape
    nblk = n // _BLK

    ahat, dinv = pl.pallas_call(
        _prep_body,
        grid=(nblk,),
        in_specs=[pl.BlockSpec((_BLK, n), lambda i: (i, 0))],
        out_specs=[
            pl.BlockSpec((_BLK, n), lambda i: (i, 0)),
            pl.BlockSpec((n, 1), lambda i: (0, 0)),
        ],
        out_shape=[
            jax.ShapeDtypeStruct((n, n), jnp.bfloat16),
            jax.ShapeDtypeStruct((n, 1), jnp.float32),
        ],
        scratch_shapes=[pltpu.VMEM((n, 1), jnp.float32)],
        interpret=interpret,
    )(adj_matrix)

    def propagate(h, w, b):
        return pl.pallas_call(
            _prop_body,
            grid=(nblk,),
            in_specs=[
                pl.BlockSpec((n, d), lambda i: (0, 0)),
                pl.BlockSpec((d, d), lambda i: (0, 0)),
                pl.BlockSpec((1, d), lambda i: (0, 0)),
                pl.BlockSpec((n, 1), lambda i: (0, 0)),
                pl.BlockSpec((_BLK, n), lambda i: (i, 0)),
            ],
            out_specs=pl.BlockSpec((n, d), lambda i: (0, 0)),
            out_shape=jax.ShapeDtypeStruct((n, d), jnp.float32),
            scratch_shapes=[
                pltpu.VMEM((n, d), jnp.bfloat16),
                pltpu.VMEM((n, d), jnp.float32),
            ],
            interpret=interpret,
        )(h, w, b.reshape(1, d), dinv, ahat)

    h = propagate(x, W1, b1)
    return propagate(h, W2, b2)


def kernel(x, adj_matrix, W1, b1, W2, b2):
    return _gcn_block(x, adj_matrix, W1, b1, W2, b2)


# speedup vs baseline: 1.1306x; 1.1306x over previous
import functools

import jax
import jax.numpy as jnp
from jax.experimental import pallas as pl
from jax.experimental.pallas import tpu as pltpu

_BLK = 512


def _prep_body(a_ref, ahat_ref, dinv_ref, deg_acc):
    i = pl.program_id(0)
    blk, n = a_ref.shape
    a = a_ref[...]
    rows = jax.lax.broadcasted_iota(jnp.int32, (blk, n), 0) + i * blk
    cols = jax.lax.broadcasted_iota(jnp.int32, (blk, n), 1)
    ahat = jnp.where(rows == cols, 1, a).astype(jnp.bfloat16)
    ahat_ref[...] = ahat
    ones = jnp.ones((1, blk), jnp.bfloat16)
    part = jax.lax.dot_general(
        ones, ahat, (((1,), (0,)), ((), ())),
        preferred_element_type=jnp.float32)

    @pl.when(i == 0)
    def _():
        deg_acc[...] = part

    @pl.when(i > 0)
    def _():
        deg_acc[...] += part

    @pl.when(i == pl.num_programs(0) - 1)
    def _():
        deg = deg_acc[...]
        dinv_ref[...] = jnp.where(deg > 0, jax.lax.rsqrt(deg), 0.0)


def _prop_body(h_ref, w_ref, b_ref, dinv_ref, ahat_ref, out_ref, g_ref, acc_ref,
               *, h_contract, transpose_out):
    i = pl.program_id(0)
    blk = ahat_ref.shape[0]

    @pl.when(i == 0)
    def _():
        hw = jax.lax.dot_general(
            w_ref[...], h_ref[...], (((0,), (h_contract,)), ((), ())),
            preferred_element_type=jnp.float32)
        g_ref[...] = (dinv_ref[...] * hw).astype(jnp.bfloat16)

    g_blk = g_ref[:, pl.ds(i * blk, blk)]
    part = jax.lax.dot_general(
        g_blk, ahat_ref[...], (((1,), (0,)), ((), ())),
        preferred_element_type=jnp.float32)

    @pl.when(i == 0)
    def _():
        acc_ref[...] = part

    @pl.when(i > 0)
    def _():
        acc_ref[...] += part

    @pl.when(i == pl.num_programs(0) - 1)
    def _():
        res = jnp.maximum(acc_ref[...] * dinv_ref[...] + b_ref[...], 0.0)
        if transpose_out:
            out_ref[...] = res.T
        else:
            out_ref[...] = res


@jax.jit
def _gcn_block(x, adj_matrix, W1, b1, W2, b2):
    n, d = x.shape
    nblk = n // _BLK

    ahat, dinv = pl.pallas_call(
        _prep_body,
        grid=(nblk,),
        in_specs=[pl.BlockSpec((_BLK, n), lambda i: (i, 0))],
        out_specs=[
            pl.BlockSpec((_BLK, n), lambda i: (i, 0)),
            pl.BlockSpec((1, n), lambda i: (0, 0)),
        ],
        out_shape=[
            jax.ShapeDtypeStruct((n, n), jnp.bfloat16),
            jax.ShapeDtypeStruct((1, n), jnp.float32),
        ],
        scratch_shapes=[pltpu.VMEM((1, n), jnp.float32)],
    )(adj_matrix)

    def propagate(h, w, b, h_contract, transpose_out, out_shape):
        return pl.pallas_call(
            functools.partial(_prop_body, h_contract=h_contract,
                              transpose_out=transpose_out),
            grid=(nblk,),
            in_specs=[
                pl.BlockSpec(h.shape, lambda i: (0, 0)),
                pl.BlockSpec((d, d), lambda i: (0, 0)),
                pl.BlockSpec((d, 1), lambda i: (0, 0)),
                pl.BlockSpec((1, n), lambda i: (0, 0)),
                pl.BlockSpec((_BLK, n), lambda i: (i, 0)),
            ],
            out_specs=pl.BlockSpec(out_shape, lambda i: (0, 0)),
            out_shape=jax.ShapeDtypeStruct(out_shape, jnp.float32),
            scratch_shapes=[
                pltpu.VMEM((d, n), jnp.bfloat16),
                pltpu.VMEM((d, n), jnp.float32),
            ],
        )(h, w, b.reshape(d, 1), dinv, ahat)

    h = propagate(x, W1, b1, h_contract=1, transpose_out=False,
                  out_shape=(d, n))
    return propagate(h, W2, b2, h_contract=0, transpose_out=True,
                     out_shape=(n, d))


def kernel(x, adj_matrix, W1, b1, W2, b2):
    return _gcn_block(x, adj_matrix, W1, b1, W2, b2)


# speedup vs baseline: 2.0241x; 1.7903x over previous
import jax
import jax.numpy as jnp
from jax.experimental import pallas as pl
from jax.experimental.pallas import tpu as pltpu

_CB = 256
_MB = 512


def _fused_body(x_ref, w1_ref, b1_ref, w2_ref, b2_ref, a_hbm, out_ref,
                ahat, stage, sem):
    n = out_ref.shape[0]
    d = out_ref.shape[1]
    nchunks = n // _CB

    copies = [
        pltpu.make_async_copy(a_hbm.at[pl.ds(k * _CB, _CB), :],
                              stage.at[k % 2], sem.at[k % 2])
        for k in range(nchunks)
    ]
    copies[0].start()
    ones = jnp.ones((1, _CB), jnp.bfloat16)
    deg = jnp.zeros((1, n), jnp.float32)
    for k in range(nchunks):
        if k + 1 < nchunks:
            copies[k + 1].start()
        copies[k].wait()
        a = stage[k % 2]
        rows = jax.lax.broadcasted_iota(jnp.int32, (_CB, n), 0) + k * _CB
        cols = jax.lax.broadcasted_iota(jnp.int32, (_CB, n), 1)
        ablk = jnp.where(rows == cols, 1, a).astype(jnp.bfloat16)
        ahat[pl.ds(k * _CB, _CB), :] = ablk
        deg += jax.lax.dot_general(
            ones, ablk, (((1,), (0,)), ((), ())),
            preferred_element_type=jnp.float32)

    dinv = jnp.where(deg > 0, jax.lax.rsqrt(deg), 0.0)

    def propagate(h, w_ref, b_ref, h_contract):
        hw = jax.lax.dot_general(
            w_ref[...], h, (((0,), (h_contract,)), ((), ())),
            preferred_element_type=jnp.float32)
        g = (dinv * hw).astype(jnp.bfloat16)
        acc = jnp.zeros((d, n), jnp.float32)
        for m in range(n // _MB):
            acc += jax.lax.dot_general(
                g[:, m * _MB:(m + 1) * _MB], ahat[pl.ds(m * _MB, _MB), :],
                (((1,), (0,)), ((), ())),
                preferred_element_type=jnp.float32)
        return jnp.maximum(acc * dinv + b_ref[...], 0.0)

    h1 = propagate(x_ref[...], w1_ref, b1_ref, 1)
    h2 = propagate(h1, w2_ref, b2_ref, 0)
    out_ref[...] = h2.T


@jax.jit
def _gcn_block(x, adj_matrix, W1, b1, W2, b2):
    n, d = x.shape
    return pl.pallas_call(
        _fused_body,
        in_specs=[
            pl.BlockSpec(memory_space=pltpu.VMEM),
            pl.BlockSpec(memory_space=pltpu.VMEM),
            pl.BlockSpec(memory_space=pltpu.VMEM),
            pl.BlockSpec(memory_space=pltpu.VMEM),
            pl.BlockSpec(memory_space=pltpu.VMEM),
            pl.BlockSpec(memory_space=pl.ANY),
        ],
        out_specs=pl.BlockSpec(memory_space=pltpu.VMEM),
        out_shape=jax.ShapeDtypeStruct((n, d), jnp.float32),
        scratch_shapes=[
            pltpu.VMEM((n, n), jnp.bfloat16),
            pltpu.VMEM((2, _CB, n), jnp.int32),
            pltpu.SemaphoreType.DMA((2,)),
        ],
    )(x, W1, b1.reshape(d, 1), W2, b2.reshape(d, 1), adj_matrix)


def kernel(x, adj_matrix, W1, b1, W2, b2):
    return _gcn_block(x, adj_matrix, W1, b1, W2, b2)


# speedup vs baseline: 2.1179x; 1.0464x over previous
import jax
import jax.numpy as jnp
from jax.experimental import pallas as pl
from jax.experimental.pallas import tpu as pltpu

_CB = 128
_NBUF = 4


def _fused_body(x_ref, w1_ref, b1_ref, w2_ref, b2_ref, a_hbm, out_ref,
                ahat, stage, sem):
    n = out_ref.shape[0]
    d = out_ref.shape[1]
    nchunks = n // _CB

    copies = [
        pltpu.make_async_copy(a_hbm.at[pl.ds(k * _CB, _CB), :],
                              stage.at[k % _NBUF], sem.at[k % _NBUF])
        for k in range(nchunks)
    ]
    for k in range(_NBUF - 1):
        copies[k].start()
    ones = jnp.ones((1, _CB), jnp.bfloat16)
    deg = jnp.zeros((1, n), jnp.float32)
    for k in range(nchunks):
        if k + _NBUF - 1 < nchunks:
            copies[k + _NBUF - 1].start()
        copies[k].wait()
        a = stage[k % _NBUF]
        rows = jax.lax.broadcasted_iota(jnp.int32, (_CB, n), 0) + k * _CB
        cols = jax.lax.broadcasted_iota(jnp.int32, (_CB, n), 1)
        ablk = jnp.where(rows == cols, 1, a).astype(jnp.bfloat16)
        ahat[pl.ds(k * _CB, _CB), :] = ablk
        deg += jax.lax.dot_general(
            ones, ablk, (((1,), (0,)), ((), ())),
            preferred_element_type=jnp.float32)

    dinv = jnp.where(deg > 0, jax.lax.rsqrt(deg), 0.0)

    def propagate(h, w_ref, b_ref, h_contract):
        hw = jax.lax.dot_general(
            w_ref[...], h, (((0,), (h_contract,)), ((), ())),
            preferred_element_type=jnp.float32)
        g = (dinv * hw).astype(jnp.bfloat16)
        acc = jax.lax.dot_general(
            g, ahat[...], (((1,), (0,)), ((), ())),
            preferred_element_type=jnp.float32)
        return jnp.maximum(acc * dinv + b_ref[...], 0.0)

    h1 = propagate(x_ref[...], w1_ref, b1_ref, 1)
    h2 = propagate(h1, w2_ref, b2_ref, 0)
    out_ref[...] = h2.T


@jax.jit
def _gcn_block(x, adj_matrix, W1, b1, W2, b2):
    n, d = x.shape
    return pl.pallas_call(
        _fused_body,
        in_specs=[
            pl.BlockSpec(memory_space=pltpu.VMEM),
            pl.BlockSpec(memory_space=pltpu.VMEM),
            pl.BlockSpec(memory_space=pltpu.VMEM),
            pl.BlockSpec(memory_space=pltpu.VMEM),
            pl.BlockSpec(memory_space=pltpu.VMEM),
            pl.BlockSpec(memory_space=pl.ANY),
        ],
        out_specs=pl.BlockSpec(memory_space=pltpu.VMEM),
        out_shape=jax.ShapeDtypeStruct((n, d), jnp.float32),
        scratch_shapes=[
            pltpu.VMEM((n, n), jnp.bfloat16),
            pltpu.VMEM((_NBUF, _CB, n), jnp.int32),
            pltpu.SemaphoreType.DMA((_NBUF,)),
        ],
    )(x, W1, b1.reshape(d, 1), W2, b2.reshape(d, 1), adj_matrix)


def kernel(x, adj_matrix, W1, b1, W2, b2):
    return _gcn_block(x, adj_matrix, W1, b1, W2, b2)


# speedup vs baseline: 2.2886x; 1.0806x over previous
import jax
import jax.numpy as jnp
from jax.experimental import pallas as pl
from jax.experimental.pallas import tpu as pltpu

_CB = 128
_NBUF = 4


def _fused_body(x_ref, w1_ref, b1_ref, w2_ref, b2_ref, a_hbm, out_ref,
                ahat, stage, sem):
    n = out_ref.shape[0]
    d = out_ref.shape[1]
    nchunks = n // _CB

    copies = [
        pltpu.make_async_copy(a_hbm.at[pl.ds(k * _CB, _CB), :],
                              stage.at[k % _NBUF], sem.at[k % _NBUF])
        for k in range(nchunks)
    ]
    for k in range(_NBUF - 1):
        copies[k].start()
    ones = jnp.ones((1, _CB), jnp.bfloat16)
    deg = jnp.zeros((1, n), jnp.float32)
    for k in range(nchunks):
        if k + _NBUF - 1 < nchunks:
            copies[k + _NBUF - 1].start()
        copies[k].wait()
        a = stage[k % _NBUF]
        rows = jax.lax.broadcasted_iota(jnp.int32, (_CB, n), 0) + k * _CB
        cols = jax.lax.broadcasted_iota(jnp.int32, (_CB, n), 1)
        ablk = jnp.where(rows == cols, 1, a).astype(jnp.bfloat16)
        ahat[pl.ds(k * _CB, _CB), :] = ablk
        deg += jax.lax.dot_general(
            ones, ablk, (((1,), (0,)), ((), ())),
            preferred_element_type=jnp.float32)

    dinv = jnp.where(deg > 0, jax.lax.rsqrt(deg), 0.0)

    def propagate(h, w_ref, b_ref, h_contract):
        hw = jax.lax.dot_general(
            w_ref[...], h, (((0,), (h_contract,)), ((), ())),
            preferred_element_type=jnp.float32)
        g = (dinv * hw).astype(jnp.bfloat16)
        acc = jax.lax.dot_general(
            g, ahat[...], (((1,), (0,)), ((), ())),
            preferred_element_type=jnp.float32)
        b_col = b_ref[...].reshape(d, 1)
        return jnp.maximum(acc * dinv + b_col, 0.0)

    h1 = propagate(x_ref[...], w1_ref, b1_ref, 1)
    h2 = propagate(h1, w2_ref, b2_ref, 0)
    out_ref[...] = h2.T


@jax.jit
def _gcn_block(x, adj_matrix, W1, b1, W2, b2):
    n, d = x.shape
    return pl.pallas_call(
        _fused_body,
        in_specs=[
            pl.BlockSpec(memory_space=pltpu.VMEM),
            pl.BlockSpec(memory_space=pltpu.VMEM),
            pl.BlockSpec(memory_space=pltpu.VMEM),
            pl.BlockSpec(memory_space=pltpu.VMEM),
            pl.BlockSpec(memory_space=pltpu.VMEM),
            pl.BlockSpec(memory_space=pl.ANY),
        ],
        out_specs=pl.BlockSpec(memory_space=pltpu.VMEM),
        out_shape=jax.ShapeDtypeStruct((n, d), jnp.float32),
        scratch_shapes=[
            pltpu.VMEM((n, n), jnp.bfloat16),
            pltpu.VMEM((_NBUF, _CB, n), jnp.int32),
            pltpu.SemaphoreType.DMA((_NBUF,)),
        ],
    )(x, W1, b1.reshape(1, d), W2, b2.reshape(1, d), adj_matrix)


def kernel(x, adj_matrix, W1, b1, W2, b2):
    return _gcn_block(x, adj_matrix, W1, b1, W2, b2)


# speedup vs baseline: 2.3106x; 1.0096x over previous
import jax
import jax.numpy as jnp
from jax.experimental import pallas as pl
from jax.experimental.pallas import tpu as pltpu

_CB = 128
_NBUF = 4
_OC = 1024


def _fused_body(w1_ref, b1_ref, w2_ref, b2_ref, x_hbm, a_hbm, out_hbm,
                ahat, stage, x_vmem, out_stage, sem, x_sem, out_sem):
    n = ahat.shape[0]
    d = x_vmem.shape[1]
    nchunks = n // _CB

    copies = [
        pltpu.make_async_copy(a_hbm.at[pl.ds(k * _CB, _CB), :],
                              stage.at[k % _NBUF], sem.at[k % _NBUF])
        for k in range(nchunks)
    ]
    x_copy = pltpu.make_async_copy(x_hbm, x_vmem, x_sem)
    for k in range(_NBUF - 1):
        copies[k].start()
    x_copy.start()
    ones = jnp.ones((1, _CB), jnp.bfloat16)
    deg = jnp.zeros((1, n), jnp.float32)
    for k in range(nchunks):
        if k + _NBUF - 1 < nchunks:
            copies[k + _NBUF - 1].start()
        copies[k].wait()
        a = stage[k % _NBUF]
        rows = jax.lax.broadcasted_iota(jnp.int32, (_CB, n), 0) + k * _CB
        cols = jax.lax.broadcasted_iota(jnp.int32, (_CB, n), 1)
        ablk = jnp.where(rows == cols, 1, a).astype(jnp.bfloat16)
        ahat[pl.ds(k * _CB, _CB), :] = ablk
        deg += jax.lax.dot_general(
            ones, ablk, (((1,), (0,)), ((), ())),
            preferred_element_type=jnp.float32)

    dinv = jnp.where(deg > 0, jax.lax.rsqrt(deg), 0.0)

    def make_g(h, w_ref, h_contract):
        hw = jax.lax.dot_general(
            w_ref[...], h, (((0,), (h_contract,)), ((), ())),
            preferred_element_type=jnp.float32)
        return (dinv * hw).astype(jnp.bfloat16)

    x_copy.wait()
    g1 = make_g(x_vmem[...], w1_ref, 1)
    acc1 = jax.lax.dot_general(
        g1, ahat[...], (((1,), (0,)), ((), ())),
        preferred_element_type=jnp.float32)
    h1 = jnp.maximum(acc1 * dinv + b1_ref[...].reshape(d, 1), 0.0)

    g2 = make_g(h1, w2_ref, 0)
    b2_col = b2_ref[...].reshape(d, 1)
    nout = n // _OC
    out_copies = [
        pltpu.make_async_copy(out_stage.at[c % 2],
                              out_hbm.at[pl.ds(c * _OC, _OC), :],
                              out_sem.at[c % 2])
        for c in range(nout)
    ]
    for c in range(nout):
        acc2 = jax.lax.dot_general(
            g2, ahat[:, c * _OC:(c + 1) * _OC], (((1,), (0,)), ((), ())),
            preferred_element_type=jnp.float32)
        res = jnp.maximum(acc2 * dinv[:, c * _OC:(c + 1) * _OC] + b2_col, 0.0)
        if c >= 2:
            out_copies[c - 2].wait()
        out_stage[c % 2] = res.T
        out_copies[c].start()
    for c in range(max(nout - 2, 0), nout):
        out_copies[c].wait()


@jax.jit
def _gcn_block(x, adj_matrix, W1, b1, W2, b2):
    n, d = x.shape
    return pl.pallas_call(
        _fused_body,
        in_specs=[
            pl.BlockSpec(memory_space=pltpu.VMEM),
            pl.BlockSpec(memory_space=pltpu.VMEM),
            pl.BlockSpec(memory_space=pltpu.VMEM),
            pl.BlockSpec(memory_space=pltpu.VMEM),
            pl.BlockSpec(memory_space=pl.ANY),
            pl.BlockSpec(memory_space=pl.ANY),
        ],
        out_specs=pl.BlockSpec(memory_space=pl.ANY),
        out_shape=jax.ShapeDtypeStruct((n, d), jnp.float32),
        scratch_shapes=[
            pltpu.VMEM((n, n), jnp.bfloat16),
            pltpu.VMEM((_NBUF, _CB, n), jnp.int32),
            pltpu.VMEM((n, d), jnp.float32),
            pltpu.VMEM((2, _OC, d), jnp.float32),
            pltpu.SemaphoreType.DMA((_NBUF,)),
            pltpu.SemaphoreType.DMA,
            pltpu.SemaphoreType.DMA((2,)),
        ],
    )(W1, b1.reshape(1, d), W2, b2.reshape(1, d), x, adj_matrix)


def kernel(x, adj_matrix, W1, b1, W2, b2):
    return _gcn_block(x, adj_matrix, W1, b1, W2, b2)
